# baseline (speedup 1.0000x reference)
import functools

import jax
import jax.numpy as jnp
from jax import lax
from jax.experimental import pallas as pl
from jax.experimental.pallas import tpu as pltpu
from jax.experimental.pallas import tpu_sc as plsc

N = 10000
E = 320000
G = 512
NC, NS = 2, 16
NW = NC * NS
C = 128
CB = 128
EPT = E // NW
NCH = 80
PADE = NCH * C - EPT
SL = 2
SLC = SL * C
NSLB = NCH // SL
HNSL = 8
NH = NSLB // HNSL
NR = 10240
RPT = NR // NS
GP = 544
NPP = 12288
PPT = NPP // NW

_mesh = lambda: plsc.VectorSubcoreMesh(core_axis_name="c", subcore_axis_name="s")


def _make_deg_kernel():
    @functools.partial(
        pl.kernel,
        mesh=_mesh(),
        out_type=(
            jax.ShapeDtypeStruct((NW, RPT), jnp.float32),
            jax.ShapeDtypeStruct((NC, GP), jnp.float32),
        ),
        scratch_types=[
            pltpu.VMEM((NCH, C), jnp.int32),
            pltpu.VMEM((3, CB), jnp.int32),
            pltpu.VMEM((CB,), jnp.float32),
            pltpu.VMEM_SHARED((NR,), jnp.float32),
            pltpu.VMEM_SHARED((GP,), jnp.float32),
        ],
    )
    def deg_kernel(dstp_hbm, batp_hbm, zdeg_hbm, zcnt_hbm, ones_hbm,
                   deg_out, cnt_out,
                   dst_v, bat_v, ones_v, acc_deg, acc_cnt):
        c = lax.axis_index("c")
        s = lax.axis_index("s")
        w = c * NS + s
        pltpu.sync_copy(zdeg_hbm, acc_deg.at[pl.ds(s * RPT, RPT)])

        @pl.when(s == 0)
        def _():
            pltpu.sync_copy(zcnt_hbm, acc_cnt)

        pltpu.sync_copy(dstp_hbm.at[w], dst_v)
        pltpu.sync_copy(batp_hbm.at[w], bat_v)
        pltpu.sync_copy(ones_hbm, ones_v)
        plsc.subcore_barrier()

        def step(j, carry):
            pltpu.sync_copy(ones_v.at[pl.ds(0, C)], acc_deg.at[dst_v.at[j]], add=True)
            return carry

        lax.fori_loop(0, NCH, step, 0)
        for j in range(3):
            pltpu.sync_copy(ones_v, acc_cnt.at[bat_v.at[j]], add=True)
        plsc.subcore_barrier()
        pltpu.sync_copy(acc_deg.at[pl.ds(s * RPT, RPT)], deg_out.at[w])

        @pl.when(s == 0)
        def _():
            pltpu.sync_copy(acc_cnt, cnt_out.at[c])

    return deg_kernel


def _make_seg_kernel(F):
    @functools.partial(
        pl.kernel,
        mesh=_mesh(),
        out_type=jax.ShapeDtypeStruct((NW, RPT, F), jnp.float32),
        scratch_types=[
            pltpu.VMEM((HNSL * SLC,), jnp.int32),
            pltpu.VMEM((HNSL * SLC,), jnp.int32),
            pltpu.VMEM((SLC, F), jnp.float32),
            pltpu.VMEM_SHARED((NR, F), jnp.float32),
            pltpu.SemaphoreType.DMA,
        ],
    )
    def seg_kernel(y_hbm, srcp_hbm, dstp_hbm, z_hbm,
                   s_out, src_v, dst_v, rows_v, acc, gsem):
        c = lax.axis_index("c")
        s = lax.axis_index("s")
        w = c * NS + s
        pltpu.sync_copy(z_hbm, acc.at[pl.ds(s * RPT, RPT)])
        plsc.subcore_barrier()

        def step(b, carry):
            sl = pl.ds(b * SLC, SLC)
            pltpu.async_copy(y_hbm.at[src_v.at[sl]], rows_v, gsem).wait()
            pltpu.sync_copy(rows_v, acc.at[dst_v.at[sl]], add=True)
            return carry

        for h in range(NH):
            pltpu.sync_copy(srcp_hbm.at[w * NH + h], src_v)
            pltpu.sync_copy(dstp_hbm.at[w * NH + h], dst_v)
            lax.fori_loop(0, HNSL, step, 0)
        plsc.subcore_barrier()
        pltpu.sync_copy(acc.at[pl.ds(s * RPT, RPT)], s_out.at[w])

    return seg_kernel


def _make_pool_kernel(F):
    @functools.partial(
        pl.kernel,
        mesh=_mesh(),
        out_type=jax.ShapeDtypeStruct((NC, GP, F), jnp.float32),
        scratch_types=[
            pltpu.VMEM((3, CB), jnp.int32),
            pltpu.VMEM((PPT, F), jnp.float32),
            pltpu.VMEM_SHARED((GP, F), jnp.float32),
        ],
    )
    def pool_kernel(h_hbm, batp_hbm, z_hbm, p_out, bat_v, rows_v, acc):
        c = lax.axis_index("c")
        s = lax.axis_index("s")
        w = c * NS + s

        @pl.when(s == 0)
        def _():
            pltpu.sync_copy(z_hbm, acc)

        pltpu.sync_copy(batp_hbm.at[w], bat_v)
        pltpu.sync_copy(h_hbm.at[pl.ds(w * PPT, PPT)], rows_v)
        plsc.subcore_barrier()
        for j in range(3):
            pltpu.sync_copy(rows_v.at[pl.ds(j * CB, CB)], acc.at[bat_v.at[j]], add=True)
        plsc.subcore_barrier()

        @pl.when(s == 0)
        def _():
            pltpu.sync_copy(acc, p_out.at[c])

    return pool_kernel


_BLK = 256
_GRID = NR // _BLK


def _tc1_body(deg_ref, x_ref, w_ref, dinv_ref, y_ref):
    deg = deg_ref[0, :] + deg_ref[1, :] + 1.0
    dinv = jnp.where(deg > 0, lax.rsqrt(jnp.maximum(deg, 1e-12)), 0.0)
    dinv_ref[...] = dinv
    y_ref[...] = dinv[:, None] * jnp.dot(
        x_ref[...], w_ref[...], preferred_element_type=jnp.float32)


def _layer_body(s_ref, y_ref, dinv_ref, b_ref, w_ref, out_ref):
    dinv = dinv_ref[...]
    h = jnp.maximum(
        dinv[:, None] * (s_ref[0] + s_ref[1] + y_ref[...]) + b_ref[...][None, :], 0.0)
    out_ref[...] = dinv[:, None] * jnp.dot(
        h, w_ref[...], preferred_element_type=jnp.float32)


def _final_body(s_ref, y_ref, dinv_ref, b_ref, h_ref):
    h_ref[...] = jnp.maximum(
        dinv_ref[...][:, None] * (s_ref[0] + s_ref[1] + y_ref[...])
        + b_ref[...][None, :], 0.0)


def _head_body(p_ref, cnt_ref, w1_ref, b1_ref, w2_ref, b2_ref, out_ref):
    sums = p_ref[0, :, :64] + p_ref[1, :, :64]
    cnt = cnt_ref[0] + cnt_ref[1]
    g = sums / jnp.maximum(cnt, 1.0)[:, None]
    z = jnp.maximum(
        jnp.dot(g, w1_ref[...], preferred_element_type=jnp.float32)
        + b1_ref[...][None, :], 0.0)
    out_ref[...] = jnp.dot(
        z, w2_ref[...], preferred_element_type=jnp.float32) + b2_ref[...][None, :]


def _tc1(deg2, x_pad, W1):
    return pl.pallas_call(
        _tc1_body,
        grid=(_GRID,),
        in_specs=[
            pl.BlockSpec((NC, _BLK), lambda i: (0, i)),
            pl.BlockSpec((_BLK, 128), lambda i: (i, 0)),
            pl.BlockSpec((128, 128), lambda i: (0, 0)),
        ],
        out_specs=[
            pl.BlockSpec((_BLK,), lambda i: (i,)),
            pl.BlockSpec((_BLK, 128), lambda i: (i, 0)),
        ],
        out_shape=[
            jax.ShapeDtypeStruct((NR,), jnp.float32),
            jax.ShapeDtypeStruct((NR, 128), jnp.float32),
        ],
    )(deg2, x_pad, W1)


def _tc_layer(s2, y, dinv, b, W, fout):
    fin = y.shape[1]
    return pl.pallas_call(
        _layer_body,
        grid=(_GRID,),
        in_specs=[
            pl.BlockSpec((NC, _BLK, fin), lambda i: (0, i, 0)),
            pl.BlockSpec((_BLK, fin), lambda i: (i, 0)),
            pl.BlockSpec((_BLK,), lambda i: (i,)),
            pl.BlockSpec((fin,), lambda i: (0,)),
            pl.BlockSpec((fin, fout), lambda i: (0, 0)),
        ],
        out_specs=pl.BlockSpec((_BLK, fout), lambda i: (i, 0)),
        out_shape=jax.ShapeDtypeStruct((NR, fout), jnp.float32),
    )(s2, y, dinv, b, W)


def _tc_final(s2, y, dinv, b):
    fin = y.shape[1]
    return pl.pallas_call(
        _final_body,
        grid=(_GRID,),
        in_specs=[
            pl.BlockSpec((NC, _BLK, fin), lambda i: (0, i, 0)),
            pl.BlockSpec((_BLK, fin), lambda i: (i, 0)),
            pl.BlockSpec((_BLK,), lambda i: (i,)),
            pl.BlockSpec((fin,), lambda i: (0,)),
        ],
        out_specs=pl.BlockSpec((_BLK, fin), lambda i: (i, 0)),
        out_shape=jax.ShapeDtypeStruct((NR, fin), jnp.float32),
    )(s2, y, dinv, b)


def _tc_head(p2, cnt2, w1, b1, w2, b2):
    return pl.pallas_call(
        _head_body,
        out_shape=jax.ShapeDtypeStruct((GP, 16), jnp.float32),
    )(p2, cnt2, w1, b1, w2, b2)


def kernel(x, edge_index, batch, conv1_W, conv1_b, conv2_W, conv2_b,
           conv3_W, conv3_b, lin1_W, lin1_b, lin2_W, lin2_b):
    f32 = jnp.float32
    src = edge_index[0].reshape(NW, EPT)
    dst = edge_index[1].reshape(NW, EPT)
    srcp = jnp.pad(src, ((0, 0), (0, PADE))).reshape(NW * NH, HNSL * SLC)
    dstp_flat = jnp.pad(dst, ((0, 0), (0, PADE)), constant_values=N)
    dstp = dstp_flat.reshape(NW * NH, HNSL * SLC)
    dstp_deg = dstp_flat.reshape(NW, NCH, C)
    batp = jnp.pad(batch, (0, NPP - N), constant_values=G).reshape(NW, 3, CB)
    x_pad = jnp.pad(x, ((0, NR - N), (0, 0)))

    z128 = jnp.zeros((RPT, 128), f32)
    zdeg = jnp.zeros((RPT,), f32)
    zcnt = jnp.zeros((GP,), f32)
    zpool = jnp.zeros((GP, 128), f32)
    ones = jnp.ones((CB,), f32)
    W3p = jnp.pad(conv3_W, ((0, 0), (0, 64)))
    b3p = jnp.pad(conv3_b, (0, 64))

    deg_p, cnt_p = _make_deg_kernel()(dstp_deg, batp, zdeg, zcnt, ones)
    deg2 = deg_p.reshape(NC, NR)

    seg128 = _make_seg_kernel(128)

    dinv, y1 = _tc1(deg2, x_pad, conv1_W)
    s1 = seg128(y1, srcp, dstp, z128).reshape(NC, NR, 128)
    y2 = _tc_layer(s1, y1, dinv, conv1_b, conv2_W, 128)
    s2 = seg128(y2, srcp, dstp, z128).reshape(NC, NR, 128)
    y3 = _tc_layer(s2, y2, dinv, conv2_b, W3p, 128)
    s3 = seg128(y3, srcp, dstp, z128).reshape(NC, NR, 128)
    h = _tc_final(s3, y3, dinv, b3p)

    h_pool = jnp.pad(h[:N], ((0, NPP - N), (0, 0)))
    p = _make_pool_kernel(128)(h_pool, batp, zpool)
    out = _tc_head(p, cnt_p, lin1_W, lin1_b, lin2_W, lin2_b)
    return out[:G]


# speedup vs baseline: 1.0469x; 1.0469x over previous
import functools

import jax
import jax.numpy as jnp
from jax import lax
from jax.experimental import pallas as pl
from jax.experimental.pallas import tpu as pltpu
from jax.experimental.pallas import tpu_sc as plsc

N = 10000
E = 320000
G = 512
NC, NS = 2, 16
NW = NC * NS
C = 128
CB = 128
EPT = E // NW
NCH = 80
PADE = NCH * C - EPT
HNCH = NCH // 2
NR = 10240
RPT = NR // NS
GP = 544
NPP = 12288
PPT = NPP // NW

_mesh = lambda: plsc.VectorSubcoreMesh(core_axis_name="c", subcore_axis_name="s")


def _make_deg_kernel():
    @functools.partial(
        pl.kernel,
        mesh=_mesh(),
        out_type=(
            jax.ShapeDtypeStruct((NW, RPT), jnp.float32),
            jax.ShapeDtypeStruct((NC, GP), jnp.float32),
        ),
        scratch_types=[
            pltpu.VMEM((NCH, C), jnp.int32),
            pltpu.VMEM((3, CB), jnp.int32),
            pltpu.VMEM((CB,), jnp.float32),
            pltpu.VMEM_SHARED((NR,), jnp.float32),
            pltpu.VMEM_SHARED((GP,), jnp.float32),
        ],
    )
    def deg_kernel(dstp_hbm, batp_hbm, zdeg_hbm, zcnt_hbm, ones_hbm,
                   deg_out, cnt_out,
                   dst_v, bat_v, ones_v, acc_deg, acc_cnt):
        c = lax.axis_index("c")
        s = lax.axis_index("s")
        w = c * NS + s
        pltpu.sync_copy(zdeg_hbm, acc_deg.at[pl.ds(s * RPT, RPT)])

        @pl.when(s == 0)
        def _():
            pltpu.sync_copy(zcnt_hbm, acc_cnt)

        pltpu.sync_copy(dstp_hbm.at[w], dst_v)
        pltpu.sync_copy(batp_hbm.at[w], bat_v)
        pltpu.sync_copy(ones_hbm, ones_v)
        plsc.subcore_barrier()

        def step(j, carry):
            pltpu.sync_copy(ones_v.at[pl.ds(0, C)], acc_deg.at[dst_v.at[j]], add=True)
            return carry

        lax.fori_loop(0, NCH, step, 0)
        for j in range(3):
            pltpu.sync_copy(ones_v, acc_cnt.at[bat_v.at[j]], add=True)
        plsc.subcore_barrier()
        pltpu.sync_copy(acc_deg.at[pl.ds(s * RPT, RPT)], deg_out.at[w])

        @pl.when(s == 0)
        def _():
            pltpu.sync_copy(acc_cnt, cnt_out.at[c])

    return deg_kernel


def _make_seg_kernel(F):
    @functools.partial(
        pl.kernel,
        mesh=_mesh(),
        out_type=jax.ShapeDtypeStruct((NW, RPT, F), jnp.float32),
        scratch_types=[
            pltpu.VMEM((HNCH, C), jnp.int32),
            pltpu.VMEM((HNCH, C), jnp.int32),
            pltpu.VMEM((2, C, F), jnp.float32),
            pltpu.VMEM_SHARED((NR, F), jnp.float32),
            pltpu.SemaphoreType.DMA,
        ],
    )
    def seg_kernel(y_hbm, srcp_hbm, dstp_hbm, z_hbm,
                   s_out, src_v, dst_v, rows_v, acc, gsem):
        c = lax.axis_index("c")
        s = lax.axis_index("s")
        w = c * NS + s
        pltpu.sync_copy(z_hbm, acc.at[pl.ds(s * RPT, RPT)])
        plsc.subcore_barrier()

        def step2(b, carry):
            j = 2 * b
            pltpu.make_async_copy(
                y_hbm.at[src_v.at[j]], rows_v.at[0], gsem).wait()
            pltpu.async_copy(y_hbm.at[src_v.at[j + 1]], rows_v.at[1], gsem)
            pltpu.sync_copy(rows_v.at[0], acc.at[dst_v.at[j]], add=True)
            pltpu.make_async_copy(
                y_hbm.at[src_v.at[j + 1]], rows_v.at[1], gsem).wait()

            @pl.when(j + 2 < HNCH)
            def _():
                pltpu.async_copy(y_hbm.at[src_v.at[j + 2]], rows_v.at[0], gsem)

            pltpu.sync_copy(rows_v.at[1], acc.at[dst_v.at[j + 1]], add=True)
            return carry

        for h in range(2):
            pltpu.sync_copy(srcp_hbm.at[w, pl.ds(h * HNCH, HNCH)], src_v)
            pltpu.sync_copy(dstp_hbm.at[w, pl.ds(h * HNCH, HNCH)], dst_v)
            pltpu.async_copy(y_hbm.at[src_v.at[0]], rows_v.at[0], gsem)
            lax.fori_loop(0, HNCH // 2, step2, 0)
        plsc.subcore_barrier()
        pltpu.sync_copy(acc.at[pl.ds(s * RPT, RPT)], s_out.at[w])

    return seg_kernel


def _make_pool_kernel(F):
    @functools.partial(
        pl.kernel,
        mesh=_mesh(),
        out_type=jax.ShapeDtypeStruct((NC, GP, F), jnp.float32),
        scratch_types=[
            pltpu.VMEM((3, CB), jnp.int32),
            pltpu.VMEM((PPT, F), jnp.float32),
            pltpu.VMEM_SHARED((GP, F), jnp.float32),
        ],
    )
    def pool_kernel(h_hbm, batp_hbm, z_hbm, p_out, bat_v, rows_v, acc):
        c = lax.axis_index("c")
        s = lax.axis_index("s")
        w = c * NS + s

        @pl.when(s == 0)
        def _():
            pltpu.sync_copy(z_hbm, acc)

        pltpu.sync_copy(batp_hbm.at[w], bat_v)
        pltpu.sync_copy(h_hbm.at[pl.ds(w * PPT, PPT)], rows_v)
        plsc.subcore_barrier()
        for j in range(3):
            pltpu.sync_copy(rows_v.at[pl.ds(j * CB, CB)], acc.at[bat_v.at[j]], add=True)
        plsc.subcore_barrier()

        @pl.when(s == 0)
        def _():
            pltpu.sync_copy(acc, p_out.at[c])

    return pool_kernel


_BLK = 256
_GRID = NR // _BLK


def _tc1_body(deg_ref, x_ref, w_ref, dinv_ref, y_ref):
    deg = deg_ref[0, :] + deg_ref[1, :] + 1.0
    dinv = jnp.where(deg > 0, lax.rsqrt(jnp.maximum(deg, 1e-12)), 0.0)
    dinv_ref[...] = dinv
    y_ref[...] = dinv[:, None] * jnp.dot(
        x_ref[...], w_ref[...], preferred_element_type=jnp.float32)


def _layer_body(s_ref, y_ref, dinv_ref, b_ref, w_ref, out_ref):
    dinv = dinv_ref[...]
    h = jnp.maximum(
        dinv[:, None] * (s_ref[0] + s_ref[1] + y_ref[...]) + b_ref[...][None, :], 0.0)
    out_ref[...] = dinv[:, None] * jnp.dot(
        h, w_ref[...], preferred_element_type=jnp.float32)


def _final_body(s_ref, y_ref, dinv_ref, b_ref, h_ref):
    h_ref[...] = jnp.maximum(
        dinv_ref[...][:, None] * (s_ref[0] + s_ref[1] + y_ref[...])
        + b_ref[...][None, :], 0.0)


def _head_body(p_ref, cnt_ref, w1_ref, b1_ref, w2_ref, b2_ref, out_ref):
    sums = p_ref[0, :, :64] + p_ref[1, :, :64]
    cnt = cnt_ref[0] + cnt_ref[1]
    g = sums / jnp.maximum(cnt, 1.0)[:, None]
    z = jnp.maximum(
        jnp.dot(g, w1_ref[...], preferred_element_type=jnp.float32)
        + b1_ref[...][None, :], 0.0)
    out_ref[...] = jnp.dot(
        z, w2_ref[...], preferred_element_type=jnp.float32) + b2_ref[...][None, :]


def _tc1(deg2, x_pad, W1):
    return pl.pallas_call(
        _tc1_body,
        grid=(_GRID,),
        in_specs=[
            pl.BlockSpec((NC, _BLK), lambda i: (0, i)),
            pl.BlockSpec((_BLK, 128), lambda i: (i, 0)),
            pl.BlockSpec((128, 128), lambda i: (0, 0)),
        ],
        out_specs=[
            pl.BlockSpec((_BLK,), lambda i: (i,)),
            pl.BlockSpec((_BLK, 128), lambda i: (i, 0)),
        ],
        out_shape=[
            jax.ShapeDtypeStruct((NR,), jnp.float32),
            jax.ShapeDtypeStruct((NR, 128), jnp.float32),
        ],
    )(deg2, x_pad, W1)


def _tc_layer(s2, y, dinv, b, W, fout):
    fin = y.shape[1]
    return pl.pallas_call(
        _layer_body,
        grid=(_GRID,),
        in_specs=[
            pl.BlockSpec((NC, _BLK, fin), lambda i: (0, i, 0)),
            pl.BlockSpec((_BLK, fin), lambda i: (i, 0)),
            pl.BlockSpec((_BLK,), lambda i: (i,)),
            pl.BlockSpec((fin,), lambda i: (0,)),
            pl.BlockSpec((fin, fout), lambda i: (0, 0)),
        ],
        out_specs=pl.BlockSpec((_BLK, fout), lambda i: (i, 0)),
        out_shape=jax.ShapeDtypeStruct((NR, fout), jnp.float32),
    )(s2, y, dinv, b, W)


def _tc_final(s2, y, dinv, b):
    fin = y.shape[1]
    return pl.pallas_call(
        _final_body,
        grid=(_GRID,),
        in_specs=[
            pl.BlockSpec((NC, _BLK, fin), lambda i: (0, i, 0)),
            pl.BlockSpec((_BLK, fin), lambda i: (i, 0)),
            pl.BlockSpec((_BLK,), lambda i: (i,)),
            pl.BlockSpec((fin,), lambda i: (0,)),
        ],
        out_specs=pl.BlockSpec((_BLK, fin), lambda i: (i, 0)),
        out_shape=jax.ShapeDtypeStruct((NR, fin), jnp.float32),
    )(s2, y, dinv, b)


def _tc_head(p2, cnt2, w1, b1, w2, b2):
    return pl.pallas_call(
        _head_body,
        out_shape=jax.ShapeDtypeStruct((GP, 16), jnp.float32),
    )(p2, cnt2, w1, b1, w2, b2)


def kernel(x, edge_index, batch, conv1_W, conv1_b, conv2_W, conv2_b,
           conv3_W, conv3_b, lin1_W, lin1_b, lin2_W, lin2_b):
    f32 = jnp.float32
    src = edge_index[0].reshape(NW, EPT)
    dst = edge_index[1].reshape(NW, EPT)
    srcp = jnp.pad(src, ((0, 0), (0, PADE))).reshape(NW, NCH, C)
    dstp = jnp.pad(dst, ((0, 0), (0, PADE)), constant_values=N).reshape(NW, NCH, C)
    dstp_deg = dstp
    batp = jnp.pad(batch, (0, NPP - N), constant_values=G).reshape(NW, 3, CB)
    x_pad = jnp.pad(x, ((0, NR - N), (0, 0)))

    z128 = jnp.zeros((RPT, 128), f32)
    zdeg = jnp.zeros((RPT,), f32)
    zcnt = jnp.zeros((GP,), f32)
    zpool = jnp.zeros((GP, 128), f32)
    ones = jnp.ones((CB,), f32)
    W3p = jnp.pad(conv3_W, ((0, 0), (0, 64)))
    b3p = jnp.pad(conv3_b, (0, 64))

    deg_p, cnt_p = _make_deg_kernel()(dstp_deg, batp, zdeg, zcnt, ones)
    deg2 = deg_p.reshape(NC, NR)

    seg128 = _make_seg_kernel(128)

    dinv, y1 = _tc1(deg2, x_pad, conv1_W)
    s1 = seg128(y1, srcp, dstp, z128).reshape(NC, NR, 128)
    y2 = _tc_layer(s1, y1, dinv, conv1_b, conv2_W, 128)
    s2 = seg128(y2, srcp, dstp, z128).reshape(NC, NR, 128)
    y3 = _tc_layer(s2, y2, dinv, conv2_b, W3p, 128)
    s3 = seg128(y3, srcp, dstp, z128).reshape(NC, NR, 128)
    h = _tc_final(s3, y3, dinv, b3p)

    h_pool = jnp.pad(h[:N], ((0, NPP - N), (0, 0)))
    p = _make_pool_kernel(128)(h_pool, batp, zpool)
    out = _tc_head(p, cnt_p, lin1_W, lin1_b, lin2_W, lin2_b)
    return out[:G]


# speedup vs baseline: 1.3765x; 1.3147x over previous
import functools

import jax
import jax.numpy as jnp
from jax import lax
from jax.experimental import pallas as pl
from jax.experimental.pallas import tpu as pltpu
from jax.experimental.pallas import tpu_sc as plsc

N = 10000
E = 320000
G = 512
NC, NS = 2, 16
NW = NC * NS
C = 128
CB = 128
EPT = E // NW
NCH = 79
PADE = NCH * C - EPT
NR = 10240
RPT = NR // NS
GP = 544
NPP = 12288
PPT = NPP // NW

_mesh = lambda: plsc.VectorSubcoreMesh(core_axis_name="c", subcore_axis_name="s")


def _make_deg_kernel():
    @functools.partial(
        pl.kernel,
        mesh=_mesh(),
        out_type=(
            jax.ShapeDtypeStruct((NW, RPT), jnp.float32),
            jax.ShapeDtypeStruct((NC, GP), jnp.float32),
        ),
        scratch_types=[
            pltpu.VMEM((NCH, C), jnp.int32),
            pltpu.VMEM((3, CB), jnp.int32),
            pltpu.VMEM((CB,), jnp.float32),
            pltpu.VMEM_SHARED((NR,), jnp.float32),
            pltpu.VMEM_SHARED((GP,), jnp.float32),
        ],
    )
    def deg_kernel(dstp_hbm, batp_hbm, zdeg_hbm, zcnt_hbm, ones_hbm,
                   deg_out, cnt_out,
                   dst_v, bat_v, ones_v, acc_deg, acc_cnt):
        c = lax.axis_index("c")
        s = lax.axis_index("s")
        w = c * NS + s
        pltpu.sync_copy(zdeg_hbm, acc_deg.at[pl.ds(s * RPT, RPT)])

        @pl.when(s == 0)
        def _():
            pltpu.sync_copy(zcnt_hbm, acc_cnt)

        pltpu.sync_copy(dstp_hbm.at[w], dst_v)
        pltpu.sync_copy(batp_hbm.at[w], bat_v)
        pltpu.sync_copy(ones_hbm, ones_v)
        plsc.subcore_barrier()

        def step(j, carry):
            pltpu.sync_copy(ones_v.at[pl.ds(0, C)], acc_deg.at[dst_v.at[j]], add=True)
            return carry

        lax.fori_loop(0, NCH, step, 0)
        for j in range(3):
            pltpu.sync_copy(ones_v, acc_cnt.at[bat_v.at[j]], add=True)
        plsc.subcore_barrier()
        pltpu.sync_copy(acc_deg.at[pl.ds(s * RPT, RPT)], deg_out.at[w])

        @pl.when(s == 0)
        def _():
            pltpu.sync_copy(acc_cnt, cnt_out.at[c])

    return deg_kernel


def _make_seg_kernel(F):
    @functools.partial(
        pl.kernel,
        mesh=_mesh(),
        out_type=jax.ShapeDtypeStruct((NW, RPT, F), jnp.float32),
        scratch_types=[
            pltpu.VMEM((NCH, C), jnp.int32),
            pltpu.VMEM((NCH, C), jnp.int32),
            pltpu.VMEM((C, F), jnp.float32),
            pltpu.VMEM_SHARED((NR, F), jnp.float32),
            pltpu.SemaphoreType.DMA,
        ],
    )
    def seg_kernel(y_hbm, srcp_hbm, dstp_hbm, z_hbm,
                   s_out, src_v, dst_v, rows_v, acc, gsem):
        c = lax.axis_index("c")
        s = lax.axis_index("s")
        w = c * NS + s
        pltpu.sync_copy(z_hbm, acc.at[pl.ds(s * RPT, RPT)])
        pltpu.sync_copy(srcp_hbm.at[w], src_v)
        pltpu.sync_copy(dstp_hbm.at[w], dst_v)
        plsc.subcore_barrier()

        def step(j, carry):
            pltpu.async_copy(y_hbm.at[src_v.at[j]], rows_v, gsem).wait()
            pltpu.sync_copy(rows_v, acc.at[dst_v.at[j]], add=True)
            return carry

        lax.fori_loop(0, NCH, step, 0)
        plsc.subcore_barrier()
        pltpu.sync_copy(acc.at[pl.ds(s * RPT, RPT)], s_out.at[w])

    return seg_kernel


def _make_pool_kernel(F):
    @functools.partial(
        pl.kernel,
        mesh=_mesh(),
        out_type=jax.ShapeDtypeStruct((NC, GP, F), jnp.float32),
        scratch_types=[
            pltpu.VMEM((3, CB), jnp.int32),
            pltpu.VMEM((PPT, F), jnp.float32),
            pltpu.VMEM_SHARED((GP, F), jnp.float32),
        ],
    )
    def pool_kernel(h_hbm, batp_hbm, z_hbm, p_out, bat_v, rows_v, acc):
        c = lax.axis_index("c")
        s = lax.axis_index("s")
        w = c * NS + s

        @pl.when(s == 0)
        def _():
            pltpu.sync_copy(z_hbm, acc)

        pltpu.sync_copy(batp_hbm.at[w], bat_v)
        pltpu.sync_copy(h_hbm.at[pl.ds(w * PPT, PPT)], rows_v)
        plsc.subcore_barrier()
        for j in range(3):
            pltpu.sync_copy(rows_v.at[pl.ds(j * CB, CB)], acc.at[bat_v.at[j]], add=True)
        plsc.subcore_barrier()

        @pl.when(s == 0)
        def _():
            pltpu.sync_copy(acc, p_out.at[c])

    return pool_kernel


_BLK = 256
_GRID = NR // _BLK


def _tc1_body(deg_ref, x_ref, w_ref, dinv_ref, y_ref):
    deg = deg_ref[0, :] + deg_ref[1, :] + 1.0
    dinv = jnp.where(deg > 0, lax.rsqrt(jnp.maximum(deg, 1e-12)), 0.0)
    dinv_ref[...] = dinv
    y_ref[...] = dinv[:, None] * jnp.dot(
        x_ref[...], w_ref[...], preferred_element_type=jnp.float32)


def _layer_body(s_ref, y_ref, dinv_ref, b_ref, w_ref, out_ref):
    dinv = dinv_ref[...]
    h = jnp.maximum(
        dinv[:, None] * (s_ref[0] + s_ref[1] + y_ref[...]) + b_ref[...][None, :], 0.0)
    out_ref[...] = dinv[:, None] * jnp.dot(
        h, w_ref[...], preferred_element_type=jnp.float32)


def _final_body(s_ref, y_ref, dinv_ref, b_ref, h_ref):
    h_ref[...] = jnp.maximum(
        dinv_ref[...][:, None] * (s_ref[0] + s_ref[1] + y_ref[...])
        + b_ref[...][None, :], 0.0)


def _head_body(p_ref, cnt_ref, w1_ref, b1_ref, w2_ref, b2_ref, out_ref):
    sums = p_ref[0, :, :64] + p_ref[1, :, :64]
    cnt = cnt_ref[0] + cnt_ref[1]
    g = sums / jnp.maximum(cnt, 1.0)[:, None]
    z = jnp.maximum(
        jnp.dot(g, w1_ref[...], preferred_element_type=jnp.float32)
        + b1_ref[...][None, :], 0.0)
    out_ref[...] = jnp.dot(
        z, w2_ref[...], preferred_element_type=jnp.float32) + b2_ref[...][None, :]


def _tc1(deg2, x_pad, W1):
    return pl.pallas_call(
        _tc1_body,
        grid=(_GRID,),
        in_specs=[
            pl.BlockSpec((NC, _BLK), lambda i: (0, i)),
            pl.BlockSpec((_BLK, 128), lambda i: (i, 0)),
            pl.BlockSpec((128, 128), lambda i: (0, 0)),
        ],
        out_specs=[
            pl.BlockSpec((_BLK,), lambda i: (i,)),
            pl.BlockSpec((_BLK, 128), lambda i: (i, 0)),
        ],
        out_shape=[
            jax.ShapeDtypeStruct((NR,), jnp.float32),
            jax.ShapeDtypeStruct((NR, 128), jnp.float32),
        ],
    )(deg2, x_pad, W1)


def _tc_layer(s2, y, dinv, b, W, fout):
    fin = y.shape[1]
    return pl.pallas_call(
        _layer_body,
        grid=(_GRID,),
        in_specs=[
            pl.BlockSpec((NC, _BLK, fin), lambda i: (0, i, 0)),
            pl.BlockSpec((_BLK, fin), lambda i: (i, 0)),
            pl.BlockSpec((_BLK,), lambda i: (i,)),
            pl.BlockSpec((fin,), lambda i: (0,)),
            pl.BlockSpec((fin, fout), lambda i: (0, 0)),
        ],
        out_specs=pl.BlockSpec((_BLK, fout), lambda i: (i, 0)),
        out_shape=jax.ShapeDtypeStruct((NR, fout), jnp.float32),
    )(s2, y, dinv, b, W)


def _tc_final(s2, y, dinv, b):
    fin = y.shape[1]
    return pl.pallas_call(
        _final_body,
        grid=(_GRID,),
        in_specs=[
            pl.BlockSpec((NC, _BLK, fin), lambda i: (0, i, 0)),
            pl.BlockSpec((_BLK, fin), lambda i: (i, 0)),
            pl.BlockSpec((_BLK,), lambda i: (i,)),
            pl.BlockSpec((fin,), lambda i: (0,)),
        ],
        out_specs=pl.BlockSpec((_BLK, fin), lambda i: (i, 0)),
        out_shape=jax.ShapeDtypeStruct((NR, fin), jnp.float32),
    )(s2, y, dinv, b)


def _tc_head(p2, cnt2, w1, b1, w2, b2):
    return pl.pallas_call(
        _head_body,
        out_shape=jax.ShapeDtypeStruct((GP, 16), jnp.float32),
    )(p2, cnt2, w1, b1, w2, b2)


def kernel(x, edge_index, batch, conv1_W, conv1_b, conv2_W, conv2_b,
           conv3_W, conv3_b, lin1_W, lin1_b, lin2_W, lin2_b):
    f32 = jnp.float32
    src = edge_index[0].reshape(NW, EPT)
    dst = edge_index[1].reshape(NW, EPT)
    srcp = jnp.pad(src, ((0, 0), (0, PADE))).reshape(NW, NCH, C)
    dstp = jnp.pad(dst, ((0, 0), (0, PADE)), constant_values=N).reshape(NW, NCH, C)
    dstp_deg = dstp
    batp = jnp.pad(batch, (0, NPP - N), constant_values=G).reshape(NW, 3, CB)
    x_pad = jnp.pad(x, ((0, NR - N), (0, 0)))

    z128 = jnp.zeros((RPT, 128), f32)
    zdeg = jnp.zeros((RPT,), f32)
    zcnt = jnp.zeros((GP,), f32)
    zpool = jnp.zeros((GP, 128), f32)
    ones = jnp.ones((CB,), f32)
    W3p = jnp.pad(conv3_W, ((0, 0), (0, 64)))
    b3p = jnp.pad(conv3_b, (0, 64))

    deg_p, cnt_p = _make_deg_kernel()(dstp_deg, batp, zdeg, zcnt, ones)
    deg2 = deg_p.reshape(NC, NR)

    seg128 = _make_seg_kernel(128)

    dinv, y1 = _tc1(deg2, x_pad, conv1_W)
    s1 = seg128(y1, srcp, dstp, z128).reshape(NC, NR, 128)
    y2 = _tc_layer(s1, y1, dinv, conv1_b, conv2_W, 128)
    s2 = seg128(y2, srcp, dstp, z128).reshape(NC, NR, 128)
    y3 = _tc_layer(s2, y2, dinv, conv2_b, W3p, 128)
    s3 = seg128(y3, srcp, dstp, z128).reshape(NC, NR, 128)
    h = _tc_final(s3, y3, dinv, b3p)

    h_pool = jnp.pad(h[:N], ((0, NPP - N), (0, 0)))
    p = _make_pool_kernel(128)(h_pool, batp, zpool)
    out = _tc_head(p, cnt_p, lin1_W, lin1_b, lin2_W, lin2_b)
    return out[:G]
